# Initial kernel scaffold; baseline (speedup 1.0000x reference)
import functools

import jax
import jax.numpy as jnp
from jax import lax
from jax.experimental import pallas as pl
from jax.experimental.pallas import tpu as pltpu
from jax.experimental.pallas import tpu_sc as plsc

NC = 2
NS = 16
NW = NC * NS
L = 16
BEDGE = 80
BM = 500


def _mesh():
    return plsc.VectorSubcoreMesh(core_axis_name="c", subcore_axis_name="s")


def _sc_deg_body(ech, npad, row_h, col_h, w_h, z_h, out_h,
                 row_v, col_v, w_v, deg_v):
    wid = lax.axis_index("s") * NC + lax.axis_index("c")
    pltpu.sync_copy(z_h, deg_v)
    off = wid * ech
    pltpu.sync_copy(row_h.at[pl.ds(off, ech)], row_v)
    pltpu.sync_copy(col_h.at[pl.ds(off, ech)], col_v)
    pltpu.sync_copy(w_h.at[pl.ds(off, ech)], w_v)

    def body(i, carry):
        sl = pl.ds(i * L, L)
        r = row_v[sl]
        c = col_v[sl]
        w = w_v[sl]
        wz = jnp.where(r == c, 0.0, w)
        plsc.addupdate_scatter(deg_v, [r], wz)
        return carry

    lax.fori_loop(0, ech // L, body, 0)
    pltpu.sync_copy(deg_v, out_h.at[wid])


def _sc_deg(row, col, w, npad):
    e = row.shape[0]
    ech = e // NW
    z = jnp.zeros((npad,), jnp.float32)
    fn = pl.kernel(
        functools.partial(_sc_deg_body, ech, npad),
        out_type=jax.ShapeDtypeStruct((NW, npad), jnp.float32),
        mesh=_mesh(),
        scratch_types=[
            pltpu.VMEM((ech,), jnp.int32),
            pltpu.VMEM((ech,), jnp.int32),
            pltpu.VMEM((ech,), jnp.float32),
            pltpu.VMEM((npad,), jnp.float32),
        ],
    )
    return fn(row, col, w, z)


def _tc_dinv_body(d_ref, o_ref):
    d = jnp.sum(d_ref[...], axis=0)
    safe = jnp.where(d > 0.0, d, 1.0)
    o_ref[...] = jnp.where(d > 0.0, lax.rsqrt(safe), 0.0)


def _tc_dinv(degp, npad):
    rows = npad // 128
    degp3 = degp.reshape(NW, rows, 128)
    out = pl.pallas_call(
        _tc_dinv_body,
        grid=(rows // 8,),
        in_specs=[pl.BlockSpec((NW, 8, 128), lambda i: (0, i, 0))],
        out_specs=pl.BlockSpec((8, 128), lambda i: (i, 0)),
        out_shape=jax.ShapeDtypeStruct((rows, 128), jnp.float32),
    )(degp3)
    return out.reshape(npad)


def _sc_norm_body(ech, npad, row_h, col_h, w_h, dinv_h, out_h,
                  row_v, col_v, w_v, dv, nrm_v):
    wid = lax.axis_index("s") * NC + lax.axis_index("c")
    off = wid * ech
    pltpu.sync_copy(dinv_h, dv)
    pltpu.sync_copy(row_h.at[pl.ds(off, ech)], row_v)
    pltpu.sync_copy(col_h.at[pl.ds(off, ech)], col_v)
    pltpu.sync_copy(w_h.at[pl.ds(off, ech)], w_v)

    def body(i, carry):
        sl = pl.ds(i * L, L)
        r = row_v[sl]
        c = col_v[sl]
        w = w_v[sl]
        dr = plsc.load_gather(dv, [r])
        dc = plsc.load_gather(dv, [c])
        wz = jnp.where(r == c, 0.0, w)
        nrm_v[sl] = -(dr * wz * dc)
        return carry

    lax.fori_loop(0, ech // L, body, 0)
    pltpu.sync_copy(nrm_v, out_h.at[pl.ds(off, ech)])


def _sc_norm(row, col, w, dinv, npad):
    e = row.shape[0]
    ech = e // NW
    fn = pl.kernel(
        functools.partial(_sc_norm_body, ech, npad),
        out_type=jax.ShapeDtypeStruct((e,), jnp.float32),
        mesh=_mesh(),
        scratch_types=[
            pltpu.VMEM((ech,), jnp.int32),
            pltpu.VMEM((ech,), jnp.int32),
            pltpu.VMEM((ech,), jnp.float32),
            pltpu.VMEM((npad,), jnp.float32),
            pltpu.VMEM((ech,), jnp.float32),
        ],
    )
    return fn(row, col, w, dinv)


def _sc_prop_body(n, ech, d, row_h, col_h, nrm_h, h_h, z_h, out_h,
                  rowb, colb, nrmb, rows_v, sem, acc_sp):
    cid = lax.axis_index("c")
    sid = lax.axis_index("s")
    wid = sid * NC + cid
    rpt = n // NS
    pltpu.sync_copy(z_h, acc_sp.at[pl.ds(sid * rpt, rpt)])
    plsc.subcore_barrier()

    def blk(i, carry):
        off = wid * ech + i * BEDGE
        pltpu.sync_copy(row_h.at[pl.ds(off, BEDGE)], rowb)
        pltpu.sync_copy(col_h.at[pl.ds(off, BEDGE)], colb)
        pltpu.sync_copy(nrm_h.at[pl.ds(off, BEDGE)], nrmb)
        pltpu.async_copy(h_h.at[rowb], rows_v, sem).wait()

        def edge(e2, c2):
            s = plsc.load_gather(nrmb, [jnp.zeros((L,), jnp.int32) + e2])
            for j in range(d // L):
                sl = pl.ds(j * L, L)
                rows_v[e2, sl] = rows_v[e2, sl] * s
            return c2

        lax.fori_loop(0, BEDGE, edge, 0)
        pltpu.sync_copy(rows_v, acc_sp.at[colb], add=True)
        return carry

    lax.fori_loop(0, ech // BEDGE, blk, 0)
    plsc.subcore_barrier()
    pltpu.sync_copy(acc_sp.at[pl.ds(sid * rpt, rpt)],
                    out_h.at[pl.ds(cid * n + sid * rpt, rpt)])


def _sc_prop(h, row, col, nrm):
    n, d = h.shape
    e = row.shape[0]
    ech = e // NW
    rpt = n // NS
    z = jnp.zeros((rpt, d), jnp.float32)
    fn = pl.kernel(
        functools.partial(_sc_prop_body, n, ech, d),
        out_type=jax.ShapeDtypeStruct((NC * n, d), jnp.float32),
        mesh=_mesh(),
        scratch_types=[
            pltpu.VMEM((BEDGE,), jnp.int32),
            pltpu.VMEM((BEDGE,), jnp.int32),
            pltpu.VMEM((BEDGE,), jnp.float32),
            pltpu.VMEM((BEDGE, d), jnp.float32),
            pltpu.SemaphoreType.DMA,
            pltpu.VMEM_SHARED((n, d), jnp.float32),
        ],
    )
    return fn(row, col, nrm, h, z)


def _tc_comb_body(u_ref, p0_ref, p1_ref, w_ref, t1_ref, acc_ref):
    t1 = p0_ref[...] + p1_ref[...]
    t1_ref[...] = t1
    acc_ref[...] = (
        jnp.dot(u_ref[...], w_ref[0], preferred_element_type=jnp.float32)
        + jnp.dot(t1, w_ref[1], preferred_element_type=jnp.float32))


def _tc_comb(u, p0, p1, w):
    n, d = u.shape
    k = w.shape[0]
    grid = (n // BM,)
    blk = pl.BlockSpec((BM, d), lambda i: (i, 0))
    t1, acc = pl.pallas_call(
        _tc_comb_body,
        grid=grid,
        in_specs=[blk, blk, blk, pl.BlockSpec((k, d, d), lambda i: (0, 0, 0))],
        out_specs=[blk, blk],
        out_shape=[jax.ShapeDtypeStruct((n, d), jnp.float32),
                   jax.ShapeDtypeStruct((n, d), jnp.float32)],
    )(u, p0, p1, w)
    return t1, acc


def _tc_out_body(relu, acc_ref, u_ref, q0_ref, q1_ref, w2_ref, b_ref, o_ref):
    t2 = 2.0 * (q0_ref[...] + q1_ref[...]) - u_ref[...]
    o = (acc_ref[...]
         + jnp.dot(t2, w2_ref[...], preferred_element_type=jnp.float32)
         + b_ref[...])
    o_ref[...] = jnp.maximum(o, 0.0) if relu else o


def _tc_out(acc, u, q0, q1, w2, b, relu):
    n, d = u.shape
    blk = pl.BlockSpec((BM, d), lambda i: (i, 0))
    return pl.pallas_call(
        functools.partial(_tc_out_body, relu),
        grid=(n // BM,),
        in_specs=[blk, blk, blk, blk,
                  pl.BlockSpec((d, d), lambda i: (0, 0)),
                  pl.BlockSpec((1, d), lambda i: (0, 0))],
        out_specs=blk,
        out_shape=jax.ShapeDtypeStruct((n, d), jnp.float32),
    )(acc, u, q0, q1, w2, b.reshape(1, d))


def kernel(x, edge_index, edge_weight, W1, b1, W2, b2):
    n, d = x.shape
    row = edge_index[0]
    col = edge_index[1]
    npad = ((n + 1023) // 1024) * 1024

    degp = _sc_deg(row, col, edge_weight, npad)
    dinv = _tc_dinv(degp, npad)
    nrm = _sc_norm(row, col, edge_weight, dinv, npad)

    h = x
    for w, b, relu in ((W1, b1, True), (W2, b2, False)):
        p = _sc_prop(h, row, col, nrm)
        t1, acc = _tc_comb(h, p[:n], p[n:], w)
        q = _sc_prop(t1, row, col, nrm)
        h = _tc_out(acc, h, q[:n], q[n:], w[2], b, relu)
    return h


# speedup vs baseline: 5.2362x; 5.2362x over previous
import functools

import jax
import jax.numpy as jnp
from jax import lax
from jax.experimental import pallas as pl
from jax.experimental.pallas import tpu as pltpu
from jax.experimental.pallas import tpu_sc as plsc

NC = 2
NS = 16
NW = NC * NS
L = 16
BEDGE = 80
BM = 1000


def _mesh():
    return plsc.VectorSubcoreMesh(core_axis_name="c", subcore_axis_name="s")


_SC_PARAMS = pltpu.CompilerParams(needs_layout_passes=False)


def _sc_deg_body(ech, npad, row_h, col_h, w_h, z_h, out_h,
                 row_v, col_v, w_v, deg_v):
    wid = lax.axis_index("s") * NC + lax.axis_index("c")
    pltpu.sync_copy(z_h, deg_v)
    off = wid * ech
    pltpu.sync_copy(row_h.at[pl.ds(off, ech)], row_v)
    pltpu.sync_copy(col_h.at[pl.ds(off, ech)], col_v)
    pltpu.sync_copy(w_h.at[pl.ds(off, ech)], w_v)

    def body(i, carry):
        sl = pl.ds(i * L, L)
        r = row_v[sl]
        c = col_v[sl]
        w = w_v[sl]
        wz = jnp.where(r == c, 0.0, w)
        plsc.addupdate_scatter(deg_v, [r], wz)
        return carry

    lax.fori_loop(0, ech // L, body, 0)
    pltpu.sync_copy(deg_v, out_h.at[wid])


def _sc_deg(row, col, w, npad):
    e = row.shape[0]
    ech = e // NW
    z = jnp.zeros((npad,), jnp.float32)
    fn = pl.kernel(
        functools.partial(_sc_deg_body, ech, npad),
        out_type=jax.ShapeDtypeStruct((NW, npad), jnp.float32),
        mesh=_mesh(),
        compiler_params=_SC_PARAMS,
        scratch_types=[
            pltpu.VMEM((ech,), jnp.int32),
            pltpu.VMEM((ech,), jnp.int32),
            pltpu.VMEM((ech,), jnp.float32),
            pltpu.VMEM((npad,), jnp.float32),
        ],
    )
    return fn(row, col, w, z)


def _tc_dinv_body(d_ref, o_ref):
    d = jnp.sum(d_ref[...], axis=0)
    safe = jnp.where(d > 0.0, d, 1.0)
    o_ref[...] = jnp.where(d > 0.0, lax.rsqrt(safe), 0.0)


def _tc_dinv(degp, npad):
    rows = npad // 128
    degp3 = degp.reshape(NW, rows, 128)
    out = pl.pallas_call(
        _tc_dinv_body,
        grid=(rows // 8,),
        in_specs=[pl.BlockSpec((NW, 8, 128), lambda i: (0, i, 0))],
        out_specs=pl.BlockSpec((8, 128), lambda i: (i, 0)),
        out_shape=jax.ShapeDtypeStruct((rows, 128), jnp.float32),
    )(degp3)
    return out.reshape(npad)


def _sc_norm_body(ech, npad, row_h, col_h, w_h, dinv_h, out_h,
                  row_v, col_v, w_v, dv, nrm_v):
    wid = lax.axis_index("s") * NC + lax.axis_index("c")
    off = wid * ech
    pltpu.sync_copy(dinv_h, dv)
    pltpu.sync_copy(row_h.at[pl.ds(off, ech)], row_v)
    pltpu.sync_copy(col_h.at[pl.ds(off, ech)], col_v)
    pltpu.sync_copy(w_h.at[pl.ds(off, ech)], w_v)

    def body(i, carry):
        sl = pl.ds(i * L, L)
        r = row_v[sl]
        c = col_v[sl]
        w = w_v[sl]
        dr = plsc.load_gather(dv, [r])
        dc = plsc.load_gather(dv, [c])
        wz = jnp.where(r == c, 0.0, w)
        nrm_v[sl] = -(dr * wz * dc)
        return carry

    lax.fori_loop(0, ech // L, body, 0)
    pltpu.sync_copy(nrm_v, out_h.at[pl.ds(off, ech)])


def _sc_norm(row, col, w, dinv, npad):
    e = row.shape[0]
    ech = e // NW
    fn = pl.kernel(
        functools.partial(_sc_norm_body, ech, npad),
        out_type=jax.ShapeDtypeStruct((e,), jnp.float32),
        mesh=_mesh(),
        compiler_params=_SC_PARAMS,
        scratch_types=[
            pltpu.VMEM((ech,), jnp.int32),
            pltpu.VMEM((ech,), jnp.int32),
            pltpu.VMEM((ech,), jnp.float32),
            pltpu.VMEM((npad,), jnp.float32),
            pltpu.VMEM((ech,), jnp.float32),
        ],
    )
    return fn(row, col, w, dinv)


def _sc_prop_body(npad, ech, d, row_h, col_h, nrm_h, h_h, z_h, out_h,
                  rowb, colb, nrmb, rows_v, sem, acc_sp):
    cid = lax.axis_index("c")
    sid = lax.axis_index("s")
    wid = sid * NC + cid
    rpt = npad // NS
    pltpu.sync_copy(z_h, acc_sp.at[pl.ds(sid * rpt, rpt)])
    plsc.subcore_barrier()

    def blk(i, carry):
        off = wid * ech + i * BEDGE
        pltpu.sync_copy(row_h.at[pl.ds(off, BEDGE)], rowb)
        pltpu.sync_copy(col_h.at[pl.ds(off, BEDGE)], colb)
        pltpu.sync_copy(nrm_h.at[pl.ds(off, BEDGE)], nrmb)
        pltpu.async_copy(h_h.at[rowb], rows_v, sem).wait()

        def edge(e2, c2):
            s = plsc.load_gather(nrmb, [jnp.zeros((L,), jnp.int32) + e2])
            for j in range(d // L):
                sl = pl.ds(j * L, L)
                rows_v[e2, sl] = rows_v[e2, sl] * s
            return c2

        lax.fori_loop(0, BEDGE, edge, 0)
        pltpu.sync_copy(rows_v, acc_sp.at[colb], add=True)
        return carry

    lax.fori_loop(0, ech // BEDGE, blk, 0)
    plsc.subcore_barrier()
    pltpu.sync_copy(acc_sp.at[pl.ds(sid * rpt, rpt)],
                    out_h.at[pl.ds(cid * npad + sid * rpt, rpt)])


def _sc_prop(h, row, col, nrm, npad):
    n, d = h.shape
    e = row.shape[0]
    ech = e // NW
    rpt = npad // NS
    z = jnp.zeros((rpt, d), jnp.float32)
    fn = pl.kernel(
        functools.partial(_sc_prop_body, npad, ech, d),
        out_type=jax.ShapeDtypeStruct((NC * npad, d), jnp.float32),
        mesh=_mesh(),
        compiler_params=_SC_PARAMS,
        scratch_types=[
            pltpu.VMEM((BEDGE,), jnp.int32),
            pltpu.VMEM((BEDGE,), jnp.int32),
            pltpu.VMEM((BEDGE,), jnp.float32),
            pltpu.VMEM((BEDGE, d), jnp.float32),
            pltpu.SemaphoreType.DMA,
            pltpu.VMEM_SHARED((npad, d), jnp.float32),
        ],
    )
    return fn(row, col, nrm, h, z)


def _tc_comb_body(u_ref, p0_ref, p1_ref, w_ref, t1_ref, acc_ref):
    t1 = p0_ref[...] + p1_ref[...]
    t1_ref[...] = t1
    acc_ref[...] = (
        jnp.dot(u_ref[...], w_ref[0], preferred_element_type=jnp.float32)
        + jnp.dot(t1, w_ref[1], preferred_element_type=jnp.float32))


def _tc_comb(u, p0, p1, w):
    n, d = u.shape
    k = w.shape[0]
    grid = (n // BM,)
    blk = pl.BlockSpec((BM, d), lambda i: (i, 0))
    t1, acc = pl.pallas_call(
        _tc_comb_body,
        grid=grid,
        in_specs=[blk, blk, blk, pl.BlockSpec((k, d, d), lambda i: (0, 0, 0))],
        out_specs=[blk, blk],
        out_shape=[jax.ShapeDtypeStruct((n, d), jnp.float32),
                   jax.ShapeDtypeStruct((n, d), jnp.float32)],
    )(u, p0, p1, w)
    return t1, acc


def _tc_out_body(relu, acc_ref, u_ref, q0_ref, q1_ref, w2_ref, b_ref, o_ref):
    t2 = 2.0 * (q0_ref[...] + q1_ref[...]) - u_ref[...]
    o = (acc_ref[...]
         + jnp.dot(t2, w2_ref[...], preferred_element_type=jnp.float32)
         + b_ref[...])
    o_ref[...] = jnp.maximum(o, 0.0) if relu else o


def _tc_out(acc, u, q0, q1, w2, b, relu):
    n, d = u.shape
    blk = pl.BlockSpec((BM, d), lambda i: (i, 0))
    return pl.pallas_call(
        functools.partial(_tc_out_body, relu),
        grid=(n // BM,),
        in_specs=[blk, blk, blk, blk,
                  pl.BlockSpec((d, d), lambda i: (0, 0)),
                  pl.BlockSpec((1, d), lambda i: (0, 0))],
        out_specs=blk,
        out_shape=jax.ShapeDtypeStruct((n, d), jnp.float32),
    )(acc, u, q0, q1, w2, b.reshape(1, d))


def kernel(x, edge_index, edge_weight, W1, b1, W2, b2):
    n, d = x.shape
    row = edge_index[0]
    col = edge_index[1]
    npad = ((n + 1023) // 1024) * 1024

    degp = _sc_deg(row, col, edge_weight, npad)
    dinv = _tc_dinv(degp, npad)
    nrm = _sc_norm(row, col, edge_weight, dinv, npad)

    h = x
    for w, b, relu in ((W1, b1, True), (W2, b2, False)):
        p = _sc_prop(h, row, col, nrm, npad)
        t1, acc = _tc_comb(h, p[:n], p[npad:npad + n], w)
        q = _sc_prop(t1, row, col, nrm, npad)
        h = _tc_out(acc, h, q[:n], q[npad:npad + n], w[2], b, relu)
    return h


# speedup vs baseline: 14.5205x; 2.7731x over previous
import functools

import jax
import jax.numpy as jnp
from jax import lax
from jax.experimental import pallas as pl
from jax.experimental.pallas import tpu as pltpu
from jax.experimental.pallas import tpu_sc as plsc

NC = 2
NS = 16
NW = NC * NS
L = 16
BEDGE = 80
BM = 1000


def _mesh():
    return plsc.VectorSubcoreMesh(core_axis_name="c", subcore_axis_name="s")


_SC_PARAMS = pltpu.CompilerParams(needs_layout_passes=False)


def _sc_deg_body(ech, npad, row_h, col_h, w_h, z_h, out_h,
                 row_v, col_v, w_v, deg_v):
    wid = lax.axis_index("s") * NC + lax.axis_index("c")
    pltpu.sync_copy(z_h, deg_v)
    off = wid * ech
    pltpu.sync_copy(row_h.at[pl.ds(off, ech)], row_v)
    pltpu.sync_copy(col_h.at[pl.ds(off, ech)], col_v)
    pltpu.sync_copy(w_h.at[pl.ds(off, ech)], w_v)

    def body(i, carry):
        sl = pl.ds(i * L, L)
        r = row_v[sl]
        c = col_v[sl]
        w = w_v[sl]
        wz = jnp.where(r == c, 0.0, w)
        plsc.addupdate_scatter(deg_v, [r], wz)
        return carry

    lax.fori_loop(0, ech // L, body, 0)
    pltpu.sync_copy(deg_v, out_h.at[wid])


def _sc_deg(row, col, w, npad):
    e = row.shape[0]
    ech = e // NW
    z = jnp.zeros((npad,), jnp.float32)
    fn = pl.kernel(
        functools.partial(_sc_deg_body, ech, npad),
        out_type=jax.ShapeDtypeStruct((NW, npad), jnp.float32),
        mesh=_mesh(),
        compiler_params=_SC_PARAMS,
        scratch_types=[
            pltpu.VMEM((ech,), jnp.int32),
            pltpu.VMEM((ech,), jnp.int32),
            pltpu.VMEM((ech,), jnp.float32),
            pltpu.VMEM((npad,), jnp.float32),
        ],
    )
    return fn(row, col, w, z)


def _tc_dinv_body(d_ref, o_ref):
    d = jnp.sum(d_ref[...], axis=0)
    safe = jnp.where(d > 0.0, d, 1.0)
    o_ref[...] = jnp.where(d > 0.0, lax.rsqrt(safe), 0.0)


def _tc_dinv(degp, npad):
    rows = npad // 128
    degp3 = degp.reshape(NW, rows, 128)
    out = pl.pallas_call(
        _tc_dinv_body,
        grid=(rows // 8,),
        in_specs=[pl.BlockSpec((NW, 8, 128), lambda i: (0, i, 0))],
        out_specs=pl.BlockSpec((8, 128), lambda i: (i, 0)),
        out_shape=jax.ShapeDtypeStruct((rows, 128), jnp.float32),
    )(degp3)
    return out.reshape(npad)


def _sc_norm_body(ech, npad, row_h, col_h, w_h, dinv_h, out_h,
                  row_v, col_v, w_v, dv, nrm_v):
    wid = lax.axis_index("s") * NC + lax.axis_index("c")
    off = wid * ech
    pltpu.sync_copy(dinv_h, dv)
    pltpu.sync_copy(row_h.at[pl.ds(off, ech)], row_v)
    pltpu.sync_copy(col_h.at[pl.ds(off, ech)], col_v)
    pltpu.sync_copy(w_h.at[pl.ds(off, ech)], w_v)

    def body(i, carry):
        sl = pl.ds(i * L, L)
        r = row_v[sl]
        c = col_v[sl]
        w = w_v[sl]
        dr = plsc.load_gather(dv, [r])
        dc = plsc.load_gather(dv, [c])
        wz = jnp.where(r == c, 0.0, w)
        nrm_v[sl] = -(dr * wz * dc)
        return carry

    lax.fori_loop(0, ech // L, body, 0)
    pltpu.sync_copy(nrm_v, out_h.at[pl.ds(off, ech)])


def _sc_norm(row, col, w, dinv, npad):
    e = row.shape[0]
    ech = e // NW
    fn = pl.kernel(
        functools.partial(_sc_norm_body, ech, npad),
        out_type=jax.ShapeDtypeStruct((e,), jnp.float32),
        mesh=_mesh(),
        compiler_params=_SC_PARAMS,
        scratch_types=[
            pltpu.VMEM((ech,), jnp.int32),
            pltpu.VMEM((ech,), jnp.int32),
            pltpu.VMEM((ech,), jnp.float32),
            pltpu.VMEM((npad,), jnp.float32),
            pltpu.VMEM((ech,), jnp.float32),
        ],
    )
    return fn(row, col, w, dinv)


def _sc_prop_body(npad, ech, d, row_h, col_h, nrm_h, h_h, z_h, out_h,
                  rowv, rows_v, colb0, colb1, colb2, nrmb0, nrmb1, nrmb2,
                  sg0, sg1, sg2, ss0, ss1, ss2,
                  scc0, scc1, scc2, scn0, scn1, scn2, acc_sp):
    cid = lax.axis_index("c")
    sid = lax.axis_index("s")
    wid = sid * NC + cid
    rpt = npad // NS
    nblk = ech // BEDGE
    colb = (colb0, colb1, colb2)
    nrmb = (nrmb0, nrmb1, nrmb2)
    sg = (sg0, sg1, sg2)
    ss = (ss0, ss1, ss2)
    scc = (scc0, scc1, scc2)
    scn = (scn0, scn1, scn2)
    ebase = wid * ech

    pltpu.sync_copy(row_h.at[pl.ds(ebase, ech)], rowv)
    pltpu.sync_copy(z_h, acc_sp.at[pl.ds(sid * rpt, rpt)])
    pltpu.async_copy(col_h.at[pl.ds(ebase, BEDGE)], colb[0], scc[0])
    pltpu.async_copy(nrm_h.at[pl.ds(ebase, BEDGE)], nrmb[0], scn[0])
    pltpu.async_copy(h_h.at[rowv.at[pl.ds(0, BEDGE)]], rows_v.at[0], sg[0])
    plsc.subcore_barrier()

    def sup(s, carry):
        for b in range(3):
            nb = (b + 1) % 3
            i = s * 3 + b

            @pl.when(i + 1 < nblk)
            def _():
                @pl.when(i >= 2)
                def _():
                    pltpu.make_async_copy(
                        rows_v.at[nb], acc_sp.at[colb[nb]], ss[nb]).wait()
                off = ebase + (i + 1) * BEDGE
                pltpu.async_copy(col_h.at[pl.ds(off, BEDGE)], colb[nb],
                                 scc[nb])
                pltpu.async_copy(nrm_h.at[pl.ds(off, BEDGE)], nrmb[nb],
                                 scn[nb])
                pltpu.async_copy(
                    h_h.at[rowv.at[pl.ds((i + 1) * BEDGE, BEDGE)]],
                    rows_v.at[nb], sg[nb])

            @pl.when(i < nblk)
            def _():
                pltpu.make_async_copy(
                    h_h.at[rowv.at[pl.ds(0, BEDGE)]], rows_v.at[b],
                    sg[b]).wait()
                pltpu.make_async_copy(
                    nrm_h.at[pl.ds(0, BEDGE)], nrmb[b], scn[b]).wait()

                def edge(e2, c2):
                    s16 = plsc.load_gather(
                        nrmb[b], [jnp.zeros((L,), jnp.int32) + e2])
                    for j in range(d // L):
                        sl = pl.ds(j * L, L)
                        rows_v[b, e2, sl] = rows_v[b, e2, sl] * s16
                    return c2

                lax.fori_loop(0, BEDGE, edge, 0)
                pltpu.make_async_copy(
                    col_h.at[pl.ds(0, BEDGE)], colb[b], scc[b]).wait()
                pltpu.async_copy(rows_v.at[b], acc_sp.at[colb[b]], ss[b],
                                 add=True)
        return carry

    lax.fori_loop(0, (nblk + 2) // 3, sup, 0)
    for b in range(3):
        pltpu.make_async_copy(rows_v.at[b], acc_sp.at[colb[b]], ss[b]).wait()
    plsc.subcore_barrier()
    pltpu.sync_copy(acc_sp.at[pl.ds(sid * rpt, rpt)],
                    out_h.at[pl.ds(cid * npad + sid * rpt, rpt)])


def _sc_prop(h, row, col, nrm, npad):
    n, d = h.shape
    e = row.shape[0]
    ech = e // NW
    rpt = npad // NS
    z = jnp.zeros((rpt, d), jnp.float32)
    fn = pl.kernel(
        functools.partial(_sc_prop_body, npad, ech, d),
        out_type=jax.ShapeDtypeStruct((NC * npad, d), jnp.float32),
        mesh=_mesh(),
        compiler_params=_SC_PARAMS,
        scratch_types=[
            pltpu.VMEM((ech,), jnp.int32),
            pltpu.VMEM((3, BEDGE, d), jnp.float32),
            pltpu.VMEM((BEDGE,), jnp.int32),
            pltpu.VMEM((BEDGE,), jnp.int32),
            pltpu.VMEM((BEDGE,), jnp.int32),
            pltpu.VMEM((BEDGE,), jnp.float32),
            pltpu.VMEM((BEDGE,), jnp.float32),
            pltpu.VMEM((BEDGE,), jnp.float32),
        ] + [pltpu.SemaphoreType.DMA] * 12 + [
            pltpu.VMEM_SHARED((npad, d), jnp.float32),
        ],
    )
    return fn(row, col, nrm, h, z)


def _tc_comb_body(u_ref, p0_ref, p1_ref, w_ref, t1_ref, acc_ref):
    t1 = p0_ref[...] + p1_ref[...]
    t1_ref[...] = t1
    acc_ref[...] = (
        jnp.dot(u_ref[...], w_ref[0], preferred_element_type=jnp.float32)
        + jnp.dot(t1, w_ref[1], preferred_element_type=jnp.float32))


def _tc_comb(u, p0, p1, w):
    n, d = u.shape
    k = w.shape[0]
    grid = (n // BM,)
    blk = pl.BlockSpec((BM, d), lambda i: (i, 0))
    t1, acc = pl.pallas_call(
        _tc_comb_body,
        grid=grid,
        in_specs=[blk, blk, blk, pl.BlockSpec((k, d, d), lambda i: (0, 0, 0))],
        out_specs=[blk, blk],
        out_shape=[jax.ShapeDtypeStruct((n, d), jnp.float32),
                   jax.ShapeDtypeStruct((n, d), jnp.float32)],
    )(u, p0, p1, w)
    return t1, acc


def _tc_out_body(relu, acc_ref, u_ref, q0_ref, q1_ref, w2_ref, b_ref, o_ref):
    t2 = 2.0 * (q0_ref[...] + q1_ref[...]) - u_ref[...]
    o = (acc_ref[...]
         + jnp.dot(t2, w2_ref[...], preferred_element_type=jnp.float32)
         + b_ref[...])
    o_ref[...] = jnp.maximum(o, 0.0) if relu else o


def _tc_out(acc, u, q0, q1, w2, b, relu):
    n, d = u.shape
    blk = pl.BlockSpec((BM, d), lambda i: (i, 0))
    return pl.pallas_call(
        functools.partial(_tc_out_body, relu),
        grid=(n // BM,),
        in_specs=[blk, blk, blk, blk,
                  pl.BlockSpec((d, d), lambda i: (0, 0)),
                  pl.BlockSpec((1, d), lambda i: (0, 0))],
        out_specs=blk,
        out_shape=jax.ShapeDtypeStruct((n, d), jnp.float32),
    )(acc, u, q0, q1, w2, b.reshape(1, d))


def kernel(x, edge_index, edge_weight, W1, b1, W2, b2):
    n, d = x.shape
    row = edge_index[0]
    col = edge_index[1]
    npad = ((n + 1023) // 1024) * 1024

    degp = _sc_deg(row, col, edge_weight, npad)
    dinv = _tc_dinv(degp, npad)
    nrm = _sc_norm(row, col, edge_weight, dinv, npad)

    h = x
    for w, b, relu in ((W1, b1, True), (W2, b2, False)):
        p = _sc_prop(h, row, col, nrm, npad)
        t1, acc = _tc_comb(h, p[:n], p[npad:npad + n], w)
        q = _sc_prop(t1, row, col, nrm, npad)
        h = _tc_out(acc, h, q[:n], q[npad:npad + n], w[2], b, relu)
    return h


# speedup vs baseline: 14.7556x; 1.0162x over previous
import functools

import jax
import jax.numpy as jnp
from jax import lax
from jax.experimental import pallas as pl
from jax.experimental.pallas import tpu as pltpu
from jax.experimental.pallas import tpu_sc as plsc

NC = 2
NS = 16
NW = NC * NS
L = 16
BEDGE = 80
BM = 1000


def _mesh():
    return plsc.VectorSubcoreMesh(core_axis_name="c", subcore_axis_name="s")


_SC_PARAMS = pltpu.CompilerParams(needs_layout_passes=False)


def _sc_deg_body(ech, npad, row_h, col_h, w_h, z_h, out_h,
                 row_v, col_v, w_v, deg_v):
    wid = lax.axis_index("s") * NC + lax.axis_index("c")
    pltpu.sync_copy(z_h, deg_v)
    off = wid * ech
    pltpu.sync_copy(row_h.at[pl.ds(off, ech)], row_v)
    pltpu.sync_copy(col_h.at[pl.ds(off, ech)], col_v)
    pltpu.sync_copy(w_h.at[pl.ds(off, ech)], w_v)

    def body(i, carry):
        sl = pl.ds(i * L, L)
        r = row_v[sl]
        c = col_v[sl]
        w = w_v[sl]
        wz = jnp.where(r == c, 0.0, w)
        plsc.addupdate_scatter(deg_v, [r], wz)
        return carry

    lax.fori_loop(0, ech // L, body, 0)
    pltpu.sync_copy(deg_v, out_h.at[wid])


def _sc_deg(row, col, w, npad):
    e = row.shape[0]
    ech = e // NW
    z = jnp.zeros((npad,), jnp.float32)
    fn = pl.kernel(
        functools.partial(_sc_deg_body, ech, npad),
        out_type=jax.ShapeDtypeStruct((NW, npad), jnp.float32),
        mesh=_mesh(),
        compiler_params=_SC_PARAMS,
        scratch_types=[
            pltpu.VMEM((ech,), jnp.int32),
            pltpu.VMEM((ech,), jnp.int32),
            pltpu.VMEM((ech,), jnp.float32),
            pltpu.VMEM((npad,), jnp.float32),
        ],
    )
    return fn(row, col, w, z)


def _tc_dinv_body(d_ref, o_ref):
    d = jnp.sum(d_ref[...], axis=0)
    safe = jnp.where(d > 0.0, d, 1.0)
    o_ref[...] = jnp.where(d > 0.0, lax.rsqrt(safe), 0.0)


def _tc_dinv(degp, npad):
    rows = npad // 128
    degp3 = degp.reshape(NW, rows, 128)
    out = pl.pallas_call(
        _tc_dinv_body,
        grid=(rows // 8,),
        in_specs=[pl.BlockSpec((NW, 8, 128), lambda i: (0, i, 0))],
        out_specs=pl.BlockSpec((8, 128), lambda i: (i, 0)),
        out_shape=jax.ShapeDtypeStruct((rows, 128), jnp.float32),
    )(degp3)
    return out.reshape(npad)


def _sc_norm_body(ech, npad, row_h, col_h, w_h, dinv_h, out_h,
                  row_v, col_v, w_v, dv, nrm_v):
    wid = lax.axis_index("s") * NC + lax.axis_index("c")
    off = wid * ech
    pltpu.sync_copy(dinv_h, dv)
    pltpu.sync_copy(row_h.at[pl.ds(off, ech)], row_v)
    pltpu.sync_copy(col_h.at[pl.ds(off, ech)], col_v)
    pltpu.sync_copy(w_h.at[pl.ds(off, ech)], w_v)

    def body(i, carry):
        sl = pl.ds(i * L, L)
        r = row_v[sl]
        c = col_v[sl]
        w = w_v[sl]
        dr = plsc.load_gather(dv, [r])
        dc = plsc.load_gather(dv, [c])
        wz = jnp.where(r == c, 0.0, w)
        nrm_v[sl] = -(dr * wz * dc)
        return carry

    lax.fori_loop(0, ech // L, body, 0)
    pltpu.sync_copy(nrm_v, out_h.at[pl.ds(off, ech)])


def _sc_norm(row, col, w, dinv, npad):
    e = row.shape[0]
    ech = e // NW
    fn = pl.kernel(
        functools.partial(_sc_norm_body, ech, npad),
        out_type=jax.ShapeDtypeStruct((e,), jnp.float32),
        mesh=_mesh(),
        compiler_params=_SC_PARAMS,
        scratch_types=[
            pltpu.VMEM((ech,), jnp.int32),
            pltpu.VMEM((ech,), jnp.int32),
            pltpu.VMEM((ech,), jnp.float32),
            pltpu.VMEM((npad,), jnp.float32),
            pltpu.VMEM((ech,), jnp.float32),
        ],
    )
    return fn(row, col, w, dinv)


def _sc_prop_body(npad, ech, d, row_h, col_h, nrm_h, h_h, z_h, out_h,
                  rowv, rows_v, colb0, colb1, colb2, nrmb0, nrmb1, nrmb2,
                  sg0, sg1, sg2, ss0, ss1, ss2,
                  scc0, scc1, scc2, scn0, scn1, scn2, acc_sp):
    cid = lax.axis_index("c")
    sid = lax.axis_index("s")
    wid = sid * NC + cid
    rpt = npad // NS
    nblk = ech // BEDGE
    colb = (colb0, colb1, colb2)
    nrmb = (nrmb0, nrmb1, nrmb2)
    sg = (sg0, sg1, sg2)
    ss = (ss0, ss1, ss2)
    scc = (scc0, scc1, scc2)
    scn = (scn0, scn1, scn2)
    ebase = wid * ech

    pltpu.sync_copy(row_h.at[pl.ds(ebase, ech)], rowv)
    pltpu.sync_copy(z_h, acc_sp.at[pl.ds(sid * rpt, rpt)])
    pltpu.async_copy(col_h.at[pl.ds(ebase, BEDGE)], colb[0], scc[0])
    pltpu.async_copy(nrm_h.at[pl.ds(ebase, BEDGE)], nrmb[0], scn[0])
    pltpu.async_copy(h_h.at[rowv.at[pl.ds(0, BEDGE)]], rows_v.at[0], sg[0])
    plsc.subcore_barrier()

    def sup(s, carry):
        for b in range(3):
            nb = (b + 1) % 3
            i = s * 3 + b

            @pl.when(i + 1 < nblk)
            def _():
                @pl.when(i >= 2)
                def _():
                    pltpu.make_async_copy(
                        rows_v.at[nb], acc_sp.at[colb[nb]], ss[nb]).wait()
                off = ebase + (i + 1) * BEDGE
                pltpu.async_copy(col_h.at[pl.ds(off, BEDGE)], colb[nb],
                                 scc[nb])
                pltpu.async_copy(nrm_h.at[pl.ds(off, BEDGE)], nrmb[nb],
                                 scn[nb])
                pltpu.async_copy(
                    h_h.at[rowv.at[pl.ds((i + 1) * BEDGE, BEDGE)]],
                    rows_v.at[nb], sg[nb])

            @pl.when(i < nblk)
            def _():
                pltpu.make_async_copy(
                    h_h.at[rowv.at[pl.ds(0, BEDGE)]], rows_v.at[b],
                    sg[b]).wait()
                pltpu.make_async_copy(
                    nrm_h.at[pl.ds(0, BEDGE)], nrmb[b], scn[b]).wait()

                def edge(e2, c2):
                    s16 = plsc.load_gather(
                        nrmb[b], [jnp.zeros((L,), jnp.int32) + e2])
                    for j in range(d // L):
                        sl = pl.ds(j * L, L)
                        rows_v[b, e2, sl] = rows_v[b, e2, sl] * s16
                    return c2

                lax.fori_loop(0, BEDGE, edge, 0, unroll=8)
                pltpu.make_async_copy(
                    col_h.at[pl.ds(0, BEDGE)], colb[b], scc[b]).wait()
                pltpu.async_copy(rows_v.at[b], acc_sp.at[colb[b]], ss[b],
                                 add=True)
        return carry

    lax.fori_loop(0, (nblk + 2) // 3, sup, 0)
    for b in range(3):
        pltpu.make_async_copy(rows_v.at[b], acc_sp.at[colb[b]], ss[b]).wait()
    plsc.subcore_barrier()
    pltpu.sync_copy(acc_sp.at[pl.ds(sid * rpt, rpt)],
                    out_h.at[pl.ds(cid * npad + sid * rpt, rpt)])


def _sc_prop(h, row, col, nrm, npad):
    n, d = h.shape
    e = row.shape[0]
    ech = e // NW
    rpt = npad // NS
    z = jnp.zeros((rpt, d), jnp.float32)
    fn = pl.kernel(
        functools.partial(_sc_prop_body, npad, ech, d),
        out_type=jax.ShapeDtypeStruct((NC * npad, d), jnp.float32),
        mesh=_mesh(),
        compiler_params=_SC_PARAMS,
        scratch_types=[
            pltpu.VMEM((ech,), jnp.int32),
            pltpu.VMEM((3, BEDGE, d), jnp.float32),
            pltpu.VMEM((BEDGE,), jnp.int32),
            pltpu.VMEM((BEDGE,), jnp.int32),
            pltpu.VMEM((BEDGE,), jnp.int32),
            pltpu.VMEM((BEDGE,), jnp.float32),
            pltpu.VMEM((BEDGE,), jnp.float32),
            pltpu.VMEM((BEDGE,), jnp.float32),
        ] + [pltpu.SemaphoreType.DMA] * 12 + [
            pltpu.VMEM_SHARED((npad, d), jnp.float32),
        ],
    )
    return fn(row, col, nrm, h, z)


def _tc_comb_body(u_ref, p0_ref, p1_ref, w_ref, t1_ref, acc_ref):
    t1 = p0_ref[...] + p1_ref[...]
    t1_ref[...] = t1
    acc_ref[...] = (
        jnp.dot(u_ref[...], w_ref[0], preferred_element_type=jnp.float32)
        + jnp.dot(t1, w_ref[1], preferred_element_type=jnp.float32))


def _tc_comb(u, p0, p1, w):
    n, d = u.shape
    k = w.shape[0]
    grid = (n // BM,)
    blk = pl.BlockSpec((BM, d), lambda i: (i, 0))
    t1, acc = pl.pallas_call(
        _tc_comb_body,
        grid=grid,
        in_specs=[blk, blk, blk, pl.BlockSpec((k, d, d), lambda i: (0, 0, 0))],
        out_specs=[blk, blk],
        out_shape=[jax.ShapeDtypeStruct((n, d), jnp.float32),
                   jax.ShapeDtypeStruct((n, d), jnp.float32)],
    )(u, p0, p1, w)
    return t1, acc


def _tc_out_body(relu, acc_ref, u_ref, q0_ref, q1_ref, w2_ref, b_ref, o_ref):
    t2 = 2.0 * (q0_ref[...] + q1_ref[...]) - u_ref[...]
    o = (acc_ref[...]
         + jnp.dot(t2, w2_ref[...], preferred_element_type=jnp.float32)
         + b_ref[...])
    o_ref[...] = jnp.maximum(o, 0.0) if relu else o


def _tc_out(acc, u, q0, q1, w2, b, relu):
    n, d = u.shape
    blk = pl.BlockSpec((BM, d), lambda i: (i, 0))
    return pl.pallas_call(
        functools.partial(_tc_out_body, relu),
        grid=(n // BM,),
        in_specs=[blk, blk, blk, blk,
                  pl.BlockSpec((d, d), lambda i: (0, 0)),
                  pl.BlockSpec((1, d), lambda i: (0, 0))],
        out_specs=blk,
        out_shape=jax.ShapeDtypeStruct((n, d), jnp.float32),
    )(acc, u, q0, q1, w2, b.reshape(1, d))


def kernel(x, edge_index, edge_weight, W1, b1, W2, b2):
    n, d = x.shape
    row = edge_index[0]
    col = edge_index[1]
    npad = ((n + 1023) // 1024) * 1024

    degp = _sc_deg(row, col, edge_weight, npad)
    dinv = _tc_dinv(degp, npad)
    nrm = _sc_norm(row, col, edge_weight, dinv, npad)

    h = x
    for w, b, relu in ((W1, b1, True), (W2, b2, False)):
        p = _sc_prop(h, row, col, nrm, npad)
        t1, acc = _tc_comb(h, p[:n], p[npad:npad + n], w)
        q = _sc_prop(t1, row, col, nrm, npad)
        h = _tc_out(acc, h, q[:n], q[npad:npad + n], w[2], b, relu)
    return h
